# Initial kernel scaffold; baseline (speedup 1.0000x reference)
import jax
import jax.numpy as jnp
from jax.experimental import pallas as pl


def kernel(x, embeddings):
    raise NotImplementedError("write your pallas kernel here")


# speedup vs baseline: 1.3883x; 1.3883x over previous
import functools

import jax
import jax.numpy as jnp
import numpy as np
from jax import lax
from jax.experimental import pallas as pl
from jax.experimental.pallas import tpu as pltpu
from jax.experimental.pallas import tpu_sc as plsc

INPUT_DIM = 3
NUM_LEVELS = 16
FEATS = 2
LOG2_HASHMAP = 19
HASHMAP_SIZE = 2 ** LOG2_HASHMAP
MASK = HASHMAP_SIZE - 1
BASE_RES = 16
N_POINTS = 524288

_PRIMES_I32 = [int(x) for x in
               np.array([1958374283, 2654435761, 805459861],
                        dtype=np.uint64).astype(np.uint32).astype(np.int32)]

NW = 32
P = 512
NPW = N_POINTS // NW
CHUNKS = NPW // P
GROUPS = P // 16
DMA_ROWS = (8 * P) // 128


def _body(x_hbm, emb_hbm, out_hbm, xb, idxb, hb, rv, wb, ob, sem):
    i32 = jnp.int32
    f32 = jnp.float32
    wid = lax.axis_index("c") * 16 + lax.axis_index("s")
    iota = lax.broadcasted_iota(i32, (16,), 0)
    zero16 = jnp.zeros((16,), i32)
    one16 = jnp.full((16,), 1, i32)
    ones_f = jnp.full((16,), 1.0, f32)

    @pl.loop(jnp.int32(0), jnp.int32(CHUNKS))
    def _chunk(chunk):
        chunk = chunk.astype(jnp.int32)
        base = wid * NPW + chunk * P
        pltpu.sync_copy(x_hbm.at[pl.ds(base, P)], xb)

        @pl.loop(jnp.int32(0), jnp.int32(NUM_LEVELS))
        def _level(l):
            l = l.astype(jnp.int32)
            res_f = jnp.left_shift(i32(BASE_RES), l).astype(f32)
            loff = jnp.left_shift(l, i32(LOG2_HASHMAP))

            @pl.loop(jnp.int32(0), jnp.int32(GROUPS))
            def _hash(g):
                g = g.astype(jnp.int32)
                pvec = g * 16 + iota
                c0 = zero16
                x0 = plsc.load_gather(xb, [pvec, c0])
                x1 = plsc.load_gather(xb, [pvec, c0 + 1])
                x2 = plsc.load_gather(xb, [pvec, c0 + 2])
                pos0 = x0 * res_f
                pos1 = x1 * res_f
                pos2 = x2 * res_f
                i0 = pos0.astype(i32)
                i1 = pos1.astype(i32)
                i2 = pos2.astype(i32)
                f0 = pos0 - i0.astype(f32)
                f1 = pos1 - i1.astype(f32)
                f2 = pos2 - i2.astype(f32)
                a0 = i0 * _PRIMES_I32[0]
                a1 = i1 * _PRIMES_I32[1]
                a2 = i2 * _PRIMES_I32[2]
                am0 = (a0 & MASK) + loff
                bm0 = ((a0 + _PRIMES_I32[0]) & MASK) + loff
                am1 = a1 & MASK
                bm1 = (a1 + _PRIMES_I32[1]) & MASK
                am2 = a2 & MASK
                bm2 = (a2 + _PRIMES_I32[2]) & MASK
                t00 = am0 ^ am1
                t10 = bm0 ^ am1
                t01 = am0 ^ bm1
                t11 = bm0 ^ bm1
                h0 = t00 ^ am2
                h1 = t10 ^ am2
                h2 = t01 ^ am2
                h3 = t11 ^ am2
                h4 = t00 ^ bm2
                h5 = t10 ^ bm2
                h6 = t01 ^ bm2
                h7 = t11 ^ bm2
                hb[g, pl.ds(0 * 16, 16)] = h0
                hb[g, pl.ds(1 * 16, 16)] = h1
                hb[g, pl.ds(2 * 16, 16)] = h2
                hb[g, pl.ds(3 * 16, 16)] = h3
                hb[g, pl.ds(4 * 16, 16)] = h4
                hb[g, pl.ds(5 * 16, 16)] = h5
                hb[g, pl.ds(6 * 16, 16)] = h6
                hb[g, pl.ds(7 * 16, 16)] = h7
                idxb[g, pl.ds(0 * 16, 16)] = jnp.right_shift(h0, i32(2))
                idxb[g, pl.ds(1 * 16, 16)] = jnp.right_shift(h1, i32(2))
                idxb[g, pl.ds(2 * 16, 16)] = jnp.right_shift(h2, i32(2))
                idxb[g, pl.ds(3 * 16, 16)] = jnp.right_shift(h3, i32(2))
                idxb[g, pl.ds(4 * 16, 16)] = jnp.right_shift(h4, i32(2))
                idxb[g, pl.ds(5 * 16, 16)] = jnp.right_shift(h5, i32(2))
                idxb[g, pl.ds(6 * 16, 16)] = jnp.right_shift(h6, i32(2))
                idxb[g, pl.ds(7 * 16, 16)] = jnp.right_shift(h7, i32(2))
                g0 = ones_f - f0
                g1 = ones_f - f1
                g2 = ones_f - f2
                u00 = g0 * g1
                u10 = f0 * g1
                u01 = g0 * f1
                u11 = f0 * f1
                off = g * 16
                wb[0, pl.ds(off, 16)] = u00 * g2
                wb[1, pl.ds(off, 16)] = u10 * g2
                wb[2, pl.ds(off, 16)] = u01 * g2
                wb[3, pl.ds(off, 16)] = u11 * g2
                wb[4, pl.ds(off, 16)] = u00 * f2
                wb[5, pl.ds(off, 16)] = u10 * f2
                wb[6, pl.ds(off, 16)] = u01 * f2
                wb[7, pl.ds(off, 16)] = u11 * f2

            @pl.loop(jnp.int32(0), jnp.int32(GROUPS))
            def _gather(g):
                g = g.astype(jnp.int32)
                pltpu.async_copy(emb_hbm.at[idxb.at[g]], rv.at[g],
                                 sem).wait()

            col_f0 = 2 * l
            col_f1 = col_f0 + 1

            @pl.loop(jnp.int32(0), jnp.int32(GROUPS))
            def _accum(g):
                g = g.astype(jnp.int32)
                gvec = zero16 + g
                pvec = g * 16 + iota
                acc0 = jnp.zeros((16,), f32)
                acc1 = jnp.zeros((16,), f32)
                for c in range(8):
                    cvec = c * 16 + iota
                    w = wb[c, pl.ds(g * 16, 16)]
                    sub2 = jnp.left_shift(hb[g, pl.ds(c * 16, 16)] & 3,
                                          i32(1))
                    v0 = plsc.load_gather(rv, [gvec, cvec, sub2])
                    v1 = plsc.load_gather(rv, [gvec, cvec, sub2 + 1])
                    acc0 = acc0 + w * v0
                    acc1 = acc1 + w * v1
                plsc.store_scatter(ob, [pvec, zero16 + col_f0], acc0)
                plsc.store_scatter(ob, [pvec, zero16 + col_f1], acc1)

        pltpu.sync_copy(ob, out_hbm.at[pl.ds(base, P)])


@jax.jit
def _encode(x, emb2):
    mesh = plsc.VectorSubcoreMesh(core_axis_name="c", subcore_axis_name="s",
                                  num_cores=2, num_subcores=16)
    return pl.kernel(
        _body,
        out_type=jax.ShapeDtypeStruct((N_POINTS, NUM_LEVELS * FEATS),
                                      jnp.float32),
        mesh=mesh,
        scratch_types=[
            pltpu.VMEM((P, INPUT_DIM), jnp.float32),
            pltpu.VMEM((DMA_ROWS, 128), jnp.int32),
            pltpu.VMEM((DMA_ROWS, 128), jnp.int32),
            pltpu.VMEM((DMA_ROWS, 128, 8), jnp.float32),
            pltpu.VMEM((8, P), jnp.float32),
            pltpu.VMEM((P, NUM_LEVELS * FEATS), jnp.float32),
            pltpu.SemaphoreType.DMA,
        ],
        compiler_params=pltpu.CompilerParams(needs_layout_passes=False,
                                             use_tc_tiling_on_sc=False),
    )(x, emb2)


def kernel(x, embeddings):
    emb8 = embeddings.reshape(NUM_LEVELS * HASHMAP_SIZE * FEATS // 8, 8)
    return _encode(x, emb8)


# speedup vs baseline: 2.2292x; 1.6057x over previous
import functools

import jax
import jax.numpy as jnp
import numpy as np
from jax import lax
from jax.experimental import pallas as pl
from jax.experimental.pallas import tpu as pltpu
from jax.experimental.pallas import tpu_sc as plsc

INPUT_DIM = 3
NUM_LEVELS = 16
FEATS = 2
LOG2_HASHMAP = 19
HASHMAP_SIZE = 2 ** LOG2_HASHMAP
MASK = HASHMAP_SIZE - 1
BASE_RES = 16
N_POINTS = 524288

_PRIMES_I32 = [int(x) for x in
               np.array([1958374283, 2654435761, 805459861],
                        dtype=np.uint64).astype(np.uint32).astype(np.int32)]

NW = 32
P = 512
NPW = N_POINTS // NW
CHUNKS = NPW // P
GROUPS = P // 16
DMA_ROWS = (8 * P) // 128


def _body(x_hbm, emb_hbm, out_hbm, xb, idxb, hb, rv, wb, ob, sem):
    i32 = jnp.int32
    f32 = jnp.float32
    wid = lax.axis_index("c") * 16 + lax.axis_index("s")
    iota = lax.broadcasted_iota(i32, (16,), 0)
    zero16 = jnp.zeros((16,), i32)
    one16 = jnp.full((16,), 1, i32)
    ones_f = jnp.full((16,), 1.0, f32)

    @pl.loop(jnp.int32(0), jnp.int32(CHUNKS))
    def _chunk(chunk):
        chunk = chunk.astype(jnp.int32)
        base = wid * NPW + chunk * P
        pltpu.sync_copy(x_hbm.at[pl.ds(base, P)], xb)

        @pl.loop(jnp.int32(0), jnp.int32(NUM_LEVELS))
        def _level(l):
            l = l.astype(jnp.int32)
            res_f = jnp.left_shift(i32(BASE_RES), l).astype(f32)
            loff = jnp.left_shift(l, i32(LOG2_HASHMAP))

            @pl.loop(jnp.int32(0), jnp.int32(GROUPS))
            def _hash(g):
                g = g.astype(jnp.int32)
                pvec = g * 16 + iota
                c0 = zero16
                x0 = plsc.load_gather(xb, [pvec, c0])
                x1 = plsc.load_gather(xb, [pvec, c0 + 1])
                x2 = plsc.load_gather(xb, [pvec, c0 + 2])
                pos0 = x0 * res_f
                pos1 = x1 * res_f
                pos2 = x2 * res_f
                i0 = pos0.astype(i32)
                i1 = pos1.astype(i32)
                i2 = pos2.astype(i32)
                f0 = pos0 - i0.astype(f32)
                f1 = pos1 - i1.astype(f32)
                f2 = pos2 - i2.astype(f32)
                a0 = i0 * _PRIMES_I32[0]
                a1 = i1 * _PRIMES_I32[1]
                a2 = i2 * _PRIMES_I32[2]
                am0 = (a0 & MASK) + loff
                bm0 = ((a0 + _PRIMES_I32[0]) & MASK) + loff
                am1 = a1 & MASK
                bm1 = (a1 + _PRIMES_I32[1]) & MASK
                am2 = a2 & MASK
                bm2 = (a2 + _PRIMES_I32[2]) & MASK
                t00 = am0 ^ am1
                t10 = bm0 ^ am1
                t01 = am0 ^ bm1
                t11 = bm0 ^ bm1
                h0 = t00 ^ am2
                h1 = t10 ^ am2
                h2 = t01 ^ am2
                h3 = t11 ^ am2
                h4 = t00 ^ bm2
                h5 = t10 ^ bm2
                h6 = t01 ^ bm2
                h7 = t11 ^ bm2
                hb[g, pl.ds(0 * 16, 16)] = h0
                hb[g, pl.ds(1 * 16, 16)] = h1
                hb[g, pl.ds(2 * 16, 16)] = h2
                hb[g, pl.ds(3 * 16, 16)] = h3
                hb[g, pl.ds(4 * 16, 16)] = h4
                hb[g, pl.ds(5 * 16, 16)] = h5
                hb[g, pl.ds(6 * 16, 16)] = h6
                hb[g, pl.ds(7 * 16, 16)] = h7
                idxb[g, pl.ds(0 * 16, 16)] = jnp.right_shift(h0, i32(2))
                idxb[g, pl.ds(1 * 16, 16)] = jnp.right_shift(h1, i32(2))
                idxb[g, pl.ds(2 * 16, 16)] = jnp.right_shift(h2, i32(2))
                idxb[g, pl.ds(3 * 16, 16)] = jnp.right_shift(h3, i32(2))
                idxb[g, pl.ds(4 * 16, 16)] = jnp.right_shift(h4, i32(2))
                idxb[g, pl.ds(5 * 16, 16)] = jnp.right_shift(h5, i32(2))
                idxb[g, pl.ds(6 * 16, 16)] = jnp.right_shift(h6, i32(2))
                idxb[g, pl.ds(7 * 16, 16)] = jnp.right_shift(h7, i32(2))
                g0 = ones_f - f0
                g1 = ones_f - f1
                g2 = ones_f - f2
                u00 = g0 * g1
                u10 = f0 * g1
                u01 = g0 * f1
                u11 = f0 * f1
                off = g * 16
                wb[0, pl.ds(off, 16)] = u00 * g2
                wb[1, pl.ds(off, 16)] = u10 * g2
                wb[2, pl.ds(off, 16)] = u01 * g2
                wb[3, pl.ds(off, 16)] = u11 * g2
                wb[4, pl.ds(off, 16)] = u00 * f2
                wb[5, pl.ds(off, 16)] = u10 * f2
                wb[6, pl.ds(off, 16)] = u01 * f2
                wb[7, pl.ds(off, 16)] = u11 * f2

            @pl.loop(jnp.int32(0), jnp.int32(GROUPS))
            def _gather(g):
                g = g.astype(jnp.int32)
                pltpu.async_copy(emb_hbm.at[idxb.at[g]], rv.at[g], sem)

            @pl.loop(jnp.int32(0), jnp.int32(GROUPS))
            def _drain(g):
                g = g.astype(jnp.int32)
                pltpu.make_async_copy(emb_hbm.at[idxb.at[g]], rv.at[g],
                                      sem).wait()

            col_f0 = 2 * l
            col_f1 = col_f0 + 1

            @pl.loop(jnp.int32(0), jnp.int32(GROUPS))
            def _accum(g):
                g = g.astype(jnp.int32)
                gvec = zero16 + g
                pvec = g * 16 + iota
                acc0 = jnp.zeros((16,), f32)
                acc1 = jnp.zeros((16,), f32)
                for c in range(8):
                    cvec = c * 16 + iota
                    w = wb[c, pl.ds(g * 16, 16)]
                    sub2 = jnp.left_shift(hb[g, pl.ds(c * 16, 16)] & 3,
                                          i32(1))
                    v0 = plsc.load_gather(rv, [gvec, cvec, sub2])
                    v1 = plsc.load_gather(rv, [gvec, cvec, sub2 + 1])
                    acc0 = acc0 + w * v0
                    acc1 = acc1 + w * v1
                plsc.store_scatter(ob, [pvec, zero16 + col_f0], acc0)
                plsc.store_scatter(ob, [pvec, zero16 + col_f1], acc1)

        pltpu.sync_copy(ob, out_hbm.at[pl.ds(base, P)])


@jax.jit
def _encode(x, emb2):
    mesh = plsc.VectorSubcoreMesh(core_axis_name="c", subcore_axis_name="s",
                                  num_cores=2, num_subcores=16)
    return pl.kernel(
        _body,
        out_type=jax.ShapeDtypeStruct((N_POINTS, NUM_LEVELS * FEATS),
                                      jnp.float32),
        mesh=mesh,
        scratch_types=[
            pltpu.VMEM((P, INPUT_DIM), jnp.float32),
            pltpu.VMEM((DMA_ROWS, 128), jnp.int32),
            pltpu.VMEM((DMA_ROWS, 128), jnp.int32),
            pltpu.VMEM((DMA_ROWS, 128, 8), jnp.float32),
            pltpu.VMEM((8, P), jnp.float32),
            pltpu.VMEM((P, NUM_LEVELS * FEATS), jnp.float32),
            pltpu.SemaphoreType.DMA,
        ],
        compiler_params=pltpu.CompilerParams(needs_layout_passes=False,
                                             use_tc_tiling_on_sc=False),
    )(x, emb2)


def kernel(x, embeddings):
    emb8 = embeddings.reshape(NUM_LEVELS * HASHMAP_SIZE * FEATS // 8, 8)
    return _encode(x, emb8)


# speedup vs baseline: 4.8987x; 2.1975x over previous
import jax
import jax.numpy as jnp
import numpy as np
from jax import lax
from jax.experimental import pallas as pl
from jax.experimental.pallas import tpu as pltpu
from jax.experimental.pallas import tpu_sc as plsc

INPUT_DIM = 3
NUM_LEVELS = 16
FEATS = 2
LOG2_HASHMAP = 19
HASHMAP_SIZE = 2 ** LOG2_HASHMAP
MASK = HASHMAP_SIZE - 1
BASE_RES = 16
N_POINTS = 524288

_PRIMES_I32 = [int(x) for x in
               np.array([1958374283, 2654435761, 805459861],
                        dtype=np.uint64).astype(np.uint32).astype(np.int32)]

NW = 32
P = 512
NPW = N_POINTS // NW
CHUNKS = NPW // P
GROUPS = P // 16
DMA_ROWS = 2 * GROUPS


def _body(x_hbm, emb_hbm, out_hbm, xb, idxb, hb, rv, wb, ob, sem):
    i32 = jnp.int32
    f32 = jnp.float32
    wid = lax.axis_index("c") * 16 + lax.axis_index("s")
    iota = lax.broadcasted_iota(i32, (16,), 0)
    zero16 = jnp.zeros((16,), i32)
    ones_f = jnp.full((16,), 1.0, f32)
    cv0 = [cb * 32 + iota for cb in range(4)]
    cv1 = [cb * 32 + 16 + iota for cb in range(4)]

    @pl.loop(jnp.int32(0), jnp.int32(CHUNKS))
    def _chunk(chunk):
        chunk = chunk.astype(jnp.int32)
        base = wid * NPW + chunk * P
        pltpu.sync_copy(x_hbm.at[pl.ds(base, P)], xb)

        @pl.loop(jnp.int32(0), jnp.int32(NUM_LEVELS))
        def _level(l):
            l = l.astype(jnp.int32)
            res_f = jnp.left_shift(i32(BASE_RES), l).astype(f32)
            loff = jnp.left_shift(l, i32(LOG2_HASHMAP))

            @pl.loop(jnp.int32(0), jnp.int32(GROUPS))
            def _hash(g):
                g = g.astype(jnp.int32)
                pvec = g * 16 + iota
                c0 = zero16
                x0 = plsc.load_gather(xb, [pvec, c0])
                x1 = plsc.load_gather(xb, [pvec, c0 + 1])
                x2 = plsc.load_gather(xb, [pvec, c0 + 2])
                pos0 = x0 * res_f
                pos1 = x1 * res_f
                pos2 = x2 * res_f
                i0 = pos0.astype(i32)
                i1 = pos1.astype(i32)
                i2 = pos2.astype(i32)
                f0 = pos0 - i0.astype(f32)
                f1 = pos1 - i1.astype(f32)
                f2 = pos2 - i2.astype(f32)
                a0 = i0 * _PRIMES_I32[0]
                a1 = i1 * _PRIMES_I32[1]
                a2 = i2 * _PRIMES_I32[2]
                am0 = (a0 & MASK) + loff
                bm0 = ((a0 + _PRIMES_I32[0]) & MASK) + loff
                am1 = a1 & MASK
                bm1 = (a1 + _PRIMES_I32[1]) & MASK
                am2 = a2 & MASK
                bm2 = (a2 + _PRIMES_I32[2]) & MASK
                t00 = am0 ^ am1
                t10 = bm0 ^ am1
                t01 = am0 ^ bm1
                t11 = bm0 ^ bm1
                hs = [t00 ^ am2, t10 ^ am2, t01 ^ am2, t11 ^ am2,
                      t00 ^ bm2, t10 ^ bm2, t01 ^ bm2, t11 ^ bm2]
                for c in range(8):
                    h = hs[c]
                    hb[g, pl.ds(c * 16, 16)] = h
                    r = (jnp.left_shift(jnp.right_shift(h, i32(7)), i32(5))
                         | (jnp.right_shift(h, i32(3)) & 15))
                    row = 2 * g + (c // 4)
                    colb = (c % 4) * 32
                    idxb[row, pl.ds(colb, 16)] = r
                    idxb[row, pl.ds(colb + 16, 16)] = r + 16
                g0 = ones_f - f0
                g1 = ones_f - f1
                g2 = ones_f - f2
                u00 = g0 * g1
                u10 = f0 * g1
                u01 = g0 * f1
                u11 = f0 * f1
                off = g * 16
                wb[0, pl.ds(off, 16)] = u00 * g2
                wb[1, pl.ds(off, 16)] = u10 * g2
                wb[2, pl.ds(off, 16)] = u01 * g2
                wb[3, pl.ds(off, 16)] = u11 * g2
                wb[4, pl.ds(off, 16)] = u00 * f2
                wb[5, pl.ds(off, 16)] = u10 * f2
                wb[6, pl.ds(off, 16)] = u01 * f2
                wb[7, pl.ds(off, 16)] = u11 * f2

            @pl.loop(jnp.int32(0), jnp.int32(DMA_ROWS))
            def _gather(r):
                r = r.astype(jnp.int32)
                pltpu.async_copy(emb_hbm.at[idxb.at[r]], rv.at[r], sem)

            @pl.loop(jnp.int32(0), jnp.int32(DMA_ROWS))
            def _drain(r):
                r = r.astype(jnp.int32)
                pltpu.make_async_copy(emb_hbm.at[idxb.at[r]], rv.at[r],
                                      sem).wait()

            row_f0 = 2 * l
            row_f1 = row_f0 + 1

            @pl.loop(jnp.int32(0), jnp.int32(GROUPS))
            def _accum(g):
                g = g.astype(jnp.int32)
                r2g = zero16 + 2 * g
                r2g1 = r2g + 1
                pvec = g * 16 + iota
                acc0 = jnp.zeros((16,), f32)
                acc1 = jnp.zeros((16,), f32)
                for c in range(8):
                    w = wb[c, pl.ds(g * 16, 16)]
                    sub = hb[g, pl.ds(c * 16, 16)] & 7
                    rvec = r2g if c < 4 else r2g1
                    v0 = plsc.load_gather(rv, [rvec, cv0[c % 4], sub])
                    v1 = plsc.load_gather(rv, [rvec, cv1[c % 4], sub])
                    acc0 = acc0 + w * v0
                    acc1 = acc1 + w * v1
                plsc.store_scatter(ob, [zero16 + row_f0, pvec], acc0)
                plsc.store_scatter(ob, [zero16 + row_f1, pvec], acc1)

        pltpu.sync_copy(ob, out_hbm.at[:, pl.ds(base, P)])


@jax.jit
def _encode(x, embp):
    mesh = plsc.VectorSubcoreMesh(core_axis_name="c", subcore_axis_name="s",
                                  num_cores=2, num_subcores=16)
    return pl.kernel(
        _body,
        out_type=jax.ShapeDtypeStruct((NUM_LEVELS * FEATS, N_POINTS),
                                      jnp.float32),
        mesh=mesh,
        scratch_types=[
            pltpu.VMEM((P, INPUT_DIM), jnp.float32),
            pltpu.VMEM((DMA_ROWS, 128), jnp.int32),
            pltpu.VMEM((GROUPS, 128), jnp.int32),
            pltpu.VMEM((DMA_ROWS, 128, 8), jnp.float32),
            pltpu.VMEM((8, P), jnp.float32),
            pltpu.VMEM((NUM_LEVELS * FEATS, P), jnp.float32),
            pltpu.SemaphoreType.DMA,
        ],
        compiler_params=pltpu.CompilerParams(needs_layout_passes=False,
                                             use_tc_tiling_on_sc=False),
    )(x, embp)


def kernel(x, embeddings):
    embp = (embeddings.reshape(NUM_LEVELS, HASHMAP_SIZE // 128, 128, FEATS)
            .transpose(0, 1, 3, 2)
            .reshape(NUM_LEVELS * HASHMAP_SIZE * FEATS // 8, 8))
    return _encode(x, embp).T


# speedup vs baseline: 6.0698x; 1.2390x over previous
import jax
import jax.numpy as jnp
import numpy as np
from jax import lax
from jax.experimental import pallas as pl
from jax.experimental.pallas import tpu as pltpu
from jax.experimental.pallas import tpu_sc as plsc

INPUT_DIM = 3
NUM_LEVELS = 16
FEATS = 2
LOG2_HASHMAP = 19
HASHMAP_SIZE = 2 ** LOG2_HASHMAP
MASK = HASHMAP_SIZE - 1
BASE_RES = 16
N_POINTS = 524288

_PRIMES_I32 = [int(x) for x in
               np.array([1958374283, 2654435761, 805459861],
                        dtype=np.uint64).astype(np.uint32).astype(np.int32)]

NW = 32
P = 256
NPW = N_POINTS // NW
CHUNKS = NPW // P
GROUPS = P // 16
DMA_ROWS = 2 * GROUPS


def _body(x_hbm, emb_hbm, out_hbm, xb, idxb, hb, rv, wb, ob,
          sem0, sem1):
    i32 = jnp.int32
    f32 = jnp.float32
    wid = lax.axis_index("c") * 16 + lax.axis_index("s")
    iota = lax.broadcasted_iota(i32, (16,), 0)
    zero16 = jnp.zeros((16,), i32)
    ones_f = jnp.full((16,), 1.0, f32)
    cv0 = [cb * 32 + iota for cb in range(4)]
    cv1 = [cb * 32 + 16 + iota for cb in range(4)]
    sems = (sem0, sem1)

    @pl.loop(jnp.int32(0), jnp.int32(CHUNKS))
    def _chunk(chunk):
        chunk = chunk.astype(jnp.int32)
        base = wid * NPW + chunk * P
        pltpu.sync_copy(x_hbm.at[pl.ds(base, P)], xb)

        def hash_and_fire(l, par):
            sem = sems[par]
            par = jnp.int32(par)
            res_f = jnp.left_shift(i32(BASE_RES), l).astype(f32)
            loff = jnp.left_shift(l, i32(LOG2_HASHMAP))

            @pl.loop(jnp.int32(0), jnp.int32(GROUPS))
            def _hash(g):
                g = g.astype(jnp.int32)
                pvec = g * 16 + iota
                c0 = zero16
                x0 = plsc.load_gather(xb, [pvec, c0])
                x1 = plsc.load_gather(xb, [pvec, c0 + 1])
                x2 = plsc.load_gather(xb, [pvec, c0 + 2])
                pos0 = x0 * res_f
                pos1 = x1 * res_f
                pos2 = x2 * res_f
                i0 = pos0.astype(i32)
                i1 = pos1.astype(i32)
                i2 = pos2.astype(i32)
                f0 = pos0 - i0.astype(f32)
                f1 = pos1 - i1.astype(f32)
                f2 = pos2 - i2.astype(f32)
                a0 = i0 * _PRIMES_I32[0]
                a1 = i1 * _PRIMES_I32[1]
                a2 = i2 * _PRIMES_I32[2]
                am0 = (a0 & MASK) + loff
                bm0 = ((a0 + _PRIMES_I32[0]) & MASK) + loff
                am1 = a1 & MASK
                bm1 = (a1 + _PRIMES_I32[1]) & MASK
                am2 = a2 & MASK
                bm2 = (a2 + _PRIMES_I32[2]) & MASK
                t00 = am0 ^ am1
                t10 = bm0 ^ am1
                t01 = am0 ^ bm1
                t11 = bm0 ^ bm1
                hs = [t00 ^ am2, t10 ^ am2, t01 ^ am2, t11 ^ am2,
                      t00 ^ bm2, t10 ^ bm2, t01 ^ bm2, t11 ^ bm2]
                for c in range(8):
                    h = hs[c]
                    hb[par, g, pl.ds(c * 16, 16)] = h
                    r = (jnp.left_shift(jnp.right_shift(h, i32(7)), i32(5))
                         | (jnp.right_shift(h, i32(3)) & 15))
                    row = 2 * g + (c // 4)
                    colb = (c % 4) * 32
                    idxb[par, row, pl.ds(colb, 16)] = r
                    idxb[par, row, pl.ds(colb + 16, 16)] = r + 16
                g0 = ones_f - f0
                g1 = ones_f - f1
                g2 = ones_f - f2
                u00 = g0 * g1
                u10 = f0 * g1
                u01 = g0 * f1
                u11 = f0 * f1
                off = g * 16
                wb[par, 0, pl.ds(off, 16)] = u00 * g2
                wb[par, 1, pl.ds(off, 16)] = u10 * g2
                wb[par, 2, pl.ds(off, 16)] = u01 * g2
                wb[par, 3, pl.ds(off, 16)] = u11 * g2
                wb[par, 4, pl.ds(off, 16)] = u00 * f2
                wb[par, 5, pl.ds(off, 16)] = u10 * f2
                wb[par, 6, pl.ds(off, 16)] = u01 * f2
                wb[par, 7, pl.ds(off, 16)] = u11 * f2

            @pl.loop(jnp.int32(0), jnp.int32(DMA_ROWS))
            def _gather(r):
                r = r.astype(jnp.int32)
                pltpu.async_copy(emb_hbm.at[idxb.at[par, r]],
                                 rv.at[par, r], sem)

        def drain_and_accum(l, par):
            sem = sems[par]
            par = jnp.int32(par)

            @pl.loop(jnp.int32(0), jnp.int32(DMA_ROWS))
            def _drain(r):
                r = r.astype(jnp.int32)
                pltpu.make_async_copy(emb_hbm.at[idxb.at[par, r]],
                                      rv.at[par, r], sem).wait()

            row_f0 = 2 * l
            row_f1 = row_f0 + 1

            @pl.loop(jnp.int32(0), jnp.int32(GROUPS))
            def _accum(g):
                g = g.astype(jnp.int32)
                r2g = zero16 + 2 * g
                r2g1 = r2g + 1
                pvec = g * 16 + iota
                acc0 = jnp.zeros((16,), f32)
                acc1 = jnp.zeros((16,), f32)
                for c in range(8):
                    w = wb[par, c, pl.ds(g * 16, 16)]
                    sub = hb[par, g, pl.ds(c * 16, 16)] & 7
                    rvec = r2g if c < 4 else r2g1
                    v0 = plsc.load_gather(rv.at[par], [rvec, cv0[c % 4], sub])
                    v1 = plsc.load_gather(rv.at[par], [rvec, cv1[c % 4], sub])
                    acc0 = acc0 + w * v0
                    acc1 = acc1 + w * v1
                plsc.store_scatter(ob, [zero16 + row_f0, pvec], acc0)
                plsc.store_scatter(ob, [zero16 + row_f1, pvec], acc1)

        @pl.loop(jnp.int32(0), jnp.int32(NUM_LEVELS // 2))
        def _lvl2(i):
            i = i.astype(jnp.int32)
            l0 = 2 * i
            hash_and_fire(l0, 0)

            @pl.when(i > 0)
            def _():
                drain_and_accum(l0 - 1, 1)

            hash_and_fire(l0 + 1, 1)
            drain_and_accum(l0, 0)

        drain_and_accum(i32(NUM_LEVELS - 1), 1)
        pltpu.sync_copy(ob, out_hbm.at[:, pl.ds(base, P)])


@jax.jit
def _encode(x, embp):
    mesh = plsc.VectorSubcoreMesh(core_axis_name="c", subcore_axis_name="s",
                                  num_cores=2, num_subcores=16)
    return pl.kernel(
        _body,
        out_type=jax.ShapeDtypeStruct((NUM_LEVELS * FEATS, N_POINTS),
                                      jnp.float32),
        mesh=mesh,
        scratch_types=[
            pltpu.VMEM((P, INPUT_DIM), jnp.float32),
            pltpu.VMEM((2, DMA_ROWS, 128), jnp.int32),
            pltpu.VMEM((2, GROUPS, 128), jnp.int32),
            pltpu.VMEM((2, DMA_ROWS, 128, 8), jnp.float32),
            pltpu.VMEM((2, 8, P), jnp.float32),
            pltpu.VMEM((NUM_LEVELS * FEATS, P), jnp.float32),
            pltpu.SemaphoreType.DMA,
            pltpu.SemaphoreType.DMA,
        ],
        compiler_params=pltpu.CompilerParams(needs_layout_passes=False,
                                             use_tc_tiling_on_sc=False),
    )(x, embp)


def kernel(x, embeddings):
    embp = (embeddings.reshape(NUM_LEVELS, HASHMAP_SIZE // 128, 128, FEATS)
            .transpose(0, 1, 3, 2)
            .reshape(NUM_LEVELS * HASHMAP_SIZE * FEATS // 8, 8))
    return _encode(x, embp).T
